# baseline (speedup 1.0000x reference)
import jax
import jax.numpy as jnp
from jax.experimental import pallas as pl
from jax.experimental.pallas import tpu as pltpu

N_USERS = 8192
N_ITEMS = 2048
BETA = 0.3
EPS = 1e-12

_BU = 1024
_BN = 256


def _prep_body(x_ref, y_ref, ni_ref):
    i = pl.program_id(0)
    x = x_ref[...]
    n_u = jnp.sum(x, axis=1, keepdims=True)
    sw = jnp.sqrt(jnp.power(n_u + EPS, -BETA))
    y_ref[...] = (sw * x).astype(jnp.bfloat16)
    col = jnp.sum(x, axis=0, keepdims=True)

    @pl.when(i == 0)
    def _():
        ni_ref[...] = col

    @pl.when(i != 0)
    def _():
        ni_ref[...] += col


def _gram_body(a_ref, b_ref, o_ref):
    o_ref[...] = jax.lax.dot_general(
        a_ref[...], b_ref[...],
        dimension_numbers=(((0,), (0,)), ((), ())),
        preferred_element_type=jnp.float32)


def kernel(X):
    n_ub = N_USERS // _BU
    Y, ni = pl.pallas_call(
        _prep_body,
        grid=(n_ub,),
        in_specs=[pl.BlockSpec((_BU, N_ITEMS), lambda i: (i, 0))],
        out_specs=[
            pl.BlockSpec((_BU, N_ITEMS), lambda i: (i, 0)),
            pl.BlockSpec((1, N_ITEMS), lambda i: (0, 0)),
        ],
        out_shape=[
            jax.ShapeDtypeStruct((N_USERS, N_ITEMS), jnp.bfloat16),
            jax.ShapeDtypeStruct((1, N_ITEMS), jnp.float32),
        ],
    )(X)

    n_ib = N_ITEMS // _BN
    G = pl.pallas_call(
        _gram_body,
        grid=(n_ib, n_ib),
        in_specs=[
            pl.BlockSpec((N_USERS, _BN), lambda i, j: (0, i)),
            pl.BlockSpec((N_USERS, _BN), lambda i, j: (0, j)),
        ],
        out_specs=pl.BlockSpec((_BN, _BN), lambda i, j: (i, j)),
        out_shape=jax.ShapeDtypeStruct((N_ITEMS, N_ITEMS), jnp.float32),
    )(Y, Y)

    return (G, ni.reshape(N_ITEMS))


# speedup vs baseline: 1.6261x; 1.6261x over previous
import jax
import jax.numpy as jnp
from jax.experimental import pallas as pl
from jax.experimental.pallas import tpu as pltpu

N_USERS = 8192
N_ITEMS = 2048
BETA = 0.3
EPS = 1e-12

_BU = 1024


def _fused_body(x_ref, g_ref, ni_ref):
    k = pl.program_id(0)
    x = x_ref[...]
    n_u = jnp.sum(x, axis=1, keepdims=True)
    sw = jnp.sqrt(jnp.power(n_u + EPS, -BETA))
    y = (sw * x).astype(jnp.bfloat16)
    g = jax.lax.dot_general(
        y, y,
        dimension_numbers=(((0,), (0,)), ((), ())),
        preferred_element_type=jnp.float32)
    col = jnp.sum(x, axis=0, keepdims=True)

    @pl.when(k == 0)
    def _():
        g_ref[...] = g
        ni_ref[...] = col

    @pl.when(k != 0)
    def _():
        g_ref[...] += g
        ni_ref[...] += col


def kernel(X):
    n_ub = N_USERS // _BU
    G, ni = pl.pallas_call(
        _fused_body,
        grid=(n_ub,),
        in_specs=[pl.BlockSpec((_BU, N_ITEMS), lambda k: (k, 0))],
        out_specs=[
            pl.BlockSpec((N_ITEMS, N_ITEMS), lambda k: (0, 0)),
            pl.BlockSpec((1, N_ITEMS), lambda k: (0, 0)),
        ],
        out_shape=[
            jax.ShapeDtypeStruct((N_ITEMS, N_ITEMS), jnp.float32),
            jax.ShapeDtypeStruct((1, N_ITEMS), jnp.float32),
        ],
    )(X)
    return (G, ni.reshape(N_ITEMS))


# speedup vs baseline: 1.8738x; 1.1523x over previous
import jax
import jax.numpy as jnp
from jax.experimental import pallas as pl
from jax.experimental.pallas import tpu as pltpu

N_USERS = 8192
N_ITEMS = 2048
BETA = 0.3
EPS = 1e-12

_BU = 1024
_T = 512
_NT = N_ITEMS // _T


def _fused_body(x_ref, g_ref, ni_ref):
    k = pl.program_id(0)
    nk = pl.num_programs(0)
    x = x_ref[...]
    n_u = jnp.sum(x, axis=1, keepdims=True)
    sw = jnp.sqrt(jnp.power(n_u + EPS, -BETA))
    y = (sw * x).astype(jnp.bfloat16)
    col = jnp.sum(x, axis=0, keepdims=True)

    @pl.when(k == 0)
    def _():
        ni_ref[...] = col

    @pl.when(k != 0)
    def _():
        ni_ref[...] += col

    for i in range(_NT):
        yi = y[:, i * _T:(i + 1) * _T]
        for j in range(i, _NT):
            yj = y[:, j * _T:(j + 1) * _T]
            blk = jax.lax.dot_general(
                yi, yj,
                dimension_numbers=(((0,), (0,)), ((), ())),
                preferred_element_type=jnp.float32)
            ii, jj = pl.ds(i * _T, _T), pl.ds(j * _T, _T)

            @pl.when(k == 0)
            def _(blk=blk, ii=ii, jj=jj):
                g_ref[ii, jj] = blk

            @pl.when(k != 0)
            def _(blk=blk, ii=ii, jj=jj):
                g_ref[ii, jj] += blk

    @pl.when(k == nk - 1)
    def _():
        for i in range(_NT):
            for j in range(i + 1, _NT):
                g_ref[pl.ds(j * _T, _T), pl.ds(i * _T, _T)] = (
                    g_ref[pl.ds(i * _T, _T), pl.ds(j * _T, _T)].T)


def kernel(X):
    n_ub = N_USERS // _BU
    G, ni = pl.pallas_call(
        _fused_body,
        grid=(n_ub,),
        in_specs=[pl.BlockSpec((_BU, N_ITEMS), lambda k: (k, 0))],
        out_specs=[
            pl.BlockSpec((N_ITEMS, N_ITEMS), lambda k: (0, 0)),
            pl.BlockSpec((1, N_ITEMS), lambda k: (0, 0)),
        ],
        out_shape=[
            jax.ShapeDtypeStruct((N_ITEMS, N_ITEMS), jnp.float32),
            jax.ShapeDtypeStruct((1, N_ITEMS), jnp.float32),
        ],
    )(X)
    return (G, ni.reshape(N_ITEMS))
